# baseline (speedup 1.0000x reference)
import jax
import jax.numpy as jnp
from jax import lax
from jax.experimental import pallas as pl
from jax.experimental.pallas import tpu as pltpu
from jax.experimental.pallas import tpu_sc as plsc

NUM_CAM = 100000
B = 16384
NC = 2
NS = 16
NW = NC * NS
BPW = B // NW
CHUNKS = BPW // 16
ROWS_PER_TILE = NUM_CAM // NS

_F1 = (1.0, -1.0 / 6, 1.0 / 120, -1.0 / 5040, 1.0 / 362880, -1.0 / 39916800)
_F2 = (0.5, -1.0 / 24, 1.0 / 720, -1.0 / 40320, 1.0 / 3628800, -1.0 / 479001600)


def _horner(s, coeffs):
    acc = jnp.full((16,), coeffs[-1], jnp.float32)
    for c in reversed(coeffs[:-1]):
        acc = acc * s + c
    return acc


def _sc_body(idx_hbm, table_hbm, out_hbm, idx_v, rows_v, obuf, spmem_tab, sem):
    cid = lax.axis_index("c")
    sid = lax.axis_index("s")
    wid = sid * NC + cid
    base = wid * BPW

    with jax.named_scope("stage_table"):
        pltpu.sync_copy(
            table_hbm.at[pl.ds(sid * ROWS_PER_TILE, ROWS_PER_TILE), :],
            spmem_tab.at[pl.ds(sid * ROWS_PER_TILE, ROWS_PER_TILE), :],
        )
    with jax.named_scope("stage_idx"):
        pltpu.sync_copy(idx_hbm.at[pl.ds(base, BPW)], idx_v)
    with jax.named_scope("barrier"):
        plsc.subcore_barrier()

    with jax.named_scope("gather"):
        pltpu.async_copy(spmem_tab.at[idx_v], rows_v, sem).wait()

    compute_scope = jax.named_scope("expmap")
    compute_scope.__enter__()
    for c in range(CHUNKS):
        rid = lax.iota(jnp.int32, 16) + c * 16

        def col(k, rid=rid):
            return plsc.load_gather(rows_v, [rid, jnp.full((16,), k, jnp.int32)])

        tx, ty, tz = col(0), col(1), col(2)
        wx, wy, wz = col(3), col(4), col(5)
        nrms = wx * wx + wy * wy + wz * wz
        s = jnp.maximum(nrms, 1e-4)
        fac1 = _horner(s, _F1)
        fac2 = _horner(s, _F2)
        zero = jnp.zeros((16,), jnp.float32)
        vals = (
            fac2 * (wx * wx - nrms) + 1.0,
            fac2 * (wx * wy) - fac1 * wz,
            fac2 * (wx * wz) + fac1 * wy,
            tx,
            fac2 * (wy * wx) + fac1 * wz,
            fac2 * (wy * wy - nrms) + 1.0,
            fac2 * (wy * wz) - fac1 * wx,
            ty,
            fac2 * (wz * wx) - fac1 * wy,
            fac2 * (wz * wy) + fac1 * wx,
            fac2 * (wz * wz - nrms) + 1.0,
            tz,
            zero, zero, zero, zero,
        )
        for k, v in enumerate(vals):
            plsc.store_scatter(obuf, [rid, jnp.full((16,), k, jnp.int32)], v)

    compute_scope.__exit__(None, None, None)
    with jax.named_scope("writeback"):
        pltpu.sync_copy(obuf, out_hbm.at[pl.ds(base, BPW), :])


@jax.jit
def kernel(indices, pose_adjustment):
    idx1d = indices.astype(jnp.int32)
    table8 = jnp.pad(pose_adjustment, ((0, 0), (0, 2)))
    mesh = plsc.VectorSubcoreMesh(
        core_axis_name="c", subcore_axis_name="s", num_cores=NC, num_subcores=NS
    )
    out16 = pl.kernel(
        _sc_body,
        out_type=jax.ShapeDtypeStruct((B, 16), jnp.float32),
        mesh=mesh,
        compiler_params=pltpu.CompilerParams(
            needs_layout_passes=False, use_tc_tiling_on_sc=False
        ),
        scratch_types=[
            pltpu.VMEM((BPW,), jnp.int32),
            pltpu.VMEM((BPW, 8), jnp.float32),
            pltpu.VMEM((BPW, 16), jnp.float32),
            pltpu.MemorySpace.VMEM_SHARED((NUM_CAM, 8), jnp.float32),
            pltpu.SemaphoreType.DMA,
        ],
    )(idx1d, table8)
    return out16.reshape(B, 4, 4)


# speedup vs baseline: 1.0310x; 1.0310x over previous
import jax
import jax.numpy as jnp
from jax import lax
from jax.experimental import pallas as pl
from jax.experimental.pallas import tpu as pltpu
from jax.experimental.pallas import tpu_sc as plsc

NUM_CAM = 100000
B = 16384
NC = 2
NS = 16
NW = NC * NS
BPW = B // NW
CHUNKS = BPW // 16

_F1 = (1.0, -1.0 / 6, 1.0 / 120, -1.0 / 5040, 1.0 / 362880, -1.0 / 39916800)
_F2 = (0.5, -1.0 / 24, 1.0 / 720, -1.0 / 40320, 1.0 / 3628800, -1.0 / 479001600)


def _horner(s, coeffs):
    acc = jnp.full((16,), coeffs[-1], jnp.float32)
    for c in reversed(coeffs[:-1]):
        acc = acc * s + c
    return acc


def _sc_body(idx_hbm, table_hbm, out_hbm, idx_v, rows_v, obuf, sem):
    wid = lax.axis_index("s") * NC + lax.axis_index("c")
    base = wid * BPW

    pltpu.sync_copy(idx_hbm.at[pl.ds(base, BPW)], idx_v)

    copies = []
    for g in range(CHUNKS):
        vec = idx_v[pl.ds(g * 16, 16)]
        copies.append(
            pltpu.async_copy(
                table_hbm.at[vec],
                rows_v.at[pl.ds(g * 16, 16), :],
                sem,
            )
        )
    for cp in copies:
        cp.wait()

    for c in range(CHUNKS):
        rid = lax.iota(jnp.int32, 16) + c * 16

        def col(k, rid=rid):
            return plsc.load_gather(rows_v, [rid, jnp.full((16,), k, jnp.int32)])

        tx, ty, tz = col(0), col(1), col(2)
        wx, wy, wz = col(3), col(4), col(5)
        nrms = wx * wx + wy * wy + wz * wz
        s = jnp.maximum(nrms, 1e-4)
        fac1 = _horner(s, _F1)
        fac2 = _horner(s, _F2)
        zero = jnp.zeros((16,), jnp.float32)
        vals = (
            fac2 * (wx * wx - nrms) + 1.0,
            fac2 * (wx * wy) - fac1 * wz,
            fac2 * (wx * wz) + fac1 * wy,
            tx,
            fac2 * (wy * wx) + fac1 * wz,
            fac2 * (wy * wy - nrms) + 1.0,
            fac2 * (wy * wz) - fac1 * wx,
            ty,
            fac2 * (wz * wx) - fac1 * wy,
            fac2 * (wz * wy) + fac1 * wx,
            fac2 * (wz * wz - nrms) + 1.0,
            tz,
            zero, zero, zero, zero,
        )
        for k, v in enumerate(vals):
            plsc.store_scatter(obuf, [rid, jnp.full((16,), k, jnp.int32)], v)

    pltpu.sync_copy(obuf, out_hbm.at[pl.ds(base, BPW), :])


@jax.jit
def kernel(indices, pose_adjustment):
    idx1d = indices.astype(jnp.int32)
    table8 = jnp.pad(pose_adjustment, ((0, 0), (0, 2)))
    mesh = plsc.VectorSubcoreMesh(
        core_axis_name="c", subcore_axis_name="s", num_cores=NC, num_subcores=NS
    )
    out16 = pl.kernel(
        _sc_body,
        out_type=jax.ShapeDtypeStruct((B, 16), jnp.float32),
        mesh=mesh,
        compiler_params=pltpu.CompilerParams(
            needs_layout_passes=False, use_tc_tiling_on_sc=False
        ),
        scratch_types=[
            pltpu.VMEM((BPW,), jnp.int32),
            pltpu.VMEM((BPW, 8), jnp.float32),
            pltpu.VMEM((BPW, 16), jnp.float32),
            pltpu.SemaphoreType.DMA,
        ],
    )(idx1d, table8)
    return out16.reshape(B, 4, 4)


# speedup vs baseline: 4.3238x; 4.1939x over previous
import jax
import jax.numpy as jnp
from jax import lax
from jax.experimental import pallas as pl
from jax.experimental.pallas import tpu as pltpu
from jax.experimental.pallas import tpu_sc as plsc

NUM_CAM = 100000
B = 16384
NC = 2
NS = 16
NW = NC * NS
BPW = B // NW
CHUNKS = BPW // 16

_F1 = (1.0, -1.0 / 6, 1.0 / 120, -1.0 / 5040, 1.0 / 362880, -1.0 / 39916800)
_F2 = (0.5, -1.0 / 24, 1.0 / 720, -1.0 / 40320, 1.0 / 3628800, -1.0 / 479001600)


def _horner(s, coeffs):
    acc = jnp.full((16,), coeffs[-1], jnp.float32)
    for c in reversed(coeffs[:-1]):
        acc = acc * s + c
    return acc


def _sc_body(idx_hbm, table_hbm, out_hbm, idx_v, ilists, comp_v, obuf, sem):
    wid = lax.axis_index("s") * NC + lax.axis_index("c")
    base = wid * BPW

    pltpu.sync_copy(idx_hbm.at[pl.ds(base, BPW)], idx_v)

    for c in range(CHUNKS):
        v = idx_v[pl.ds(c * 16, 16)]
        for j in range(6):
            ilists[j, pl.ds(c * 16, 16)] = v + j * NUM_CAM

    copies = [
        pltpu.async_copy(table_hbm.at[ilists.at[j]], comp_v.at[j], sem)
        for j in range(6)
    ]
    for cp in copies:
        cp.wait()

    zero = jnp.zeros((16,), jnp.float32)
    for c in range(CHUNKS):
        sl = pl.ds(c * 16, 16)
        tx, ty, tz = comp_v[0, sl], comp_v[1, sl], comp_v[2, sl]
        wx, wy, wz = comp_v[3, sl], comp_v[4, sl], comp_v[5, sl]
        nrms = wx * wx + wy * wy + wz * wz
        s = jnp.maximum(nrms, 1e-4)
        fac1 = _horner(s, _F1)
        fac2 = _horner(s, _F2)
        vals = (
            fac2 * (wx * wx - nrms) + 1.0,
            fac2 * (wx * wy) - fac1 * wz,
            fac2 * (wx * wz) + fac1 * wy,
            tx,
            fac2 * (wy * wx) + fac1 * wz,
            fac2 * (wy * wy - nrms) + 1.0,
            fac2 * (wy * wz) - fac1 * wx,
            ty,
            fac2 * (wz * wx) - fac1 * wy,
            fac2 * (wz * wy) + fac1 * wx,
            fac2 * (wz * wz - nrms) + 1.0,
            tz,
            zero, zero, zero, zero,
        )
        for k, v in enumerate(vals):
            obuf[k, sl] = v

    pltpu.sync_copy(obuf, out_hbm.at[:, pl.ds(base, BPW)])


@jax.jit
def kernel(indices, pose_adjustment):
    idx1d = indices.astype(jnp.int32)
    tab_flat = pose_adjustment.T.reshape(-1)
    mesh = plsc.VectorSubcoreMesh(
        core_axis_name="c", subcore_axis_name="s", num_cores=NC, num_subcores=NS
    )
    out16 = pl.kernel(
        _sc_body,
        out_type=jax.ShapeDtypeStruct((16, B), jnp.float32),
        mesh=mesh,
        compiler_params=pltpu.CompilerParams(
            needs_layout_passes=False, use_tc_tiling_on_sc=False
        ),
        scratch_types=[
            pltpu.VMEM((BPW,), jnp.int32),
            pltpu.VMEM((6, BPW), jnp.int32),
            pltpu.VMEM((6, BPW), jnp.float32),
            pltpu.VMEM((16, BPW), jnp.float32),
            pltpu.SemaphoreType.DMA,
        ],
    )(idx1d, tab_flat)
    return out16.reshape(4, 4, B).transpose(2, 0, 1)
